# baseline (speedup 1.0000x reference)
import jax
import jax.numpy as jnp
from jax import lax
from jax.experimental import pallas as pl
from jax.experimental.pallas import tpu as pltpu

N_TOK = 16384
D = 4096
E = 64
K = 8
M_BLK = 1024
DH = D // 2


def _router_block(xa_ref, xb_ref, wta_ref, wtb_ref, b_ref, rw_ref, gates_ref):
    acc = jnp.dot(xa_ref[...], wta_ref[...], preferred_element_type=jnp.float32)
    acc = acc + jnp.dot(xb_ref[...], wtb_ref[...], preferred_element_type=jnp.float32)
    rw = acc + b_ref[...]
    rw_ref[...] = rw

    cur = rw
    t = jnp.max(cur, axis=1, keepdims=True)
    m0 = t
    s = jnp.ones_like(t)
    for _ in range(K - 1):
        cur = jnp.where(cur == t, -jnp.inf, cur)
        t = jnp.max(cur, axis=1, keepdims=True)
        s = s + jnp.exp(t - m0)
    rinv = 1.0 / s
    gates_ref[...] = jnp.where(rw >= t, jnp.exp(rw - m0) * rinv, 0.0)


@jax.jit
def kernel(x, W, b):
    wt = W.T
    b2 = b.reshape(1, E)
    grid = (N_TOK // M_BLK,)
    rw, gates = pl.pallas_call(
        _router_block,
        grid=grid,
        in_specs=[
            pl.BlockSpec((M_BLK, DH), lambda i: (i, 0)),
            pl.BlockSpec((M_BLK, DH), lambda i: (i, 1)),
            pl.BlockSpec((DH, E), lambda i: (0, 0)),
            pl.BlockSpec((DH, E), lambda i: (1, 0)),
            pl.BlockSpec((1, E), lambda i: (0, 0)),
        ],
        out_specs=[
            pl.BlockSpec((M_BLK, E), lambda i: (i, 0)),
            pl.BlockSpec((M_BLK, E), lambda i: (i, 0)),
        ],
        out_shape=[
            jax.ShapeDtypeStruct((N_TOK, E), jnp.float32),
            jax.ShapeDtypeStruct((N_TOK, E), jnp.float32),
        ],
        compiler_params=pltpu.CompilerParams(
            dimension_semantics=("arbitrary",),
        ),
    )(x, x, wt, wt, b2)
    return (gates, rw)


# speedup vs baseline: 1.0096x; 1.0096x over previous
import functools

import jax
import jax.numpy as jnp
from jax import lax
from jax.experimental import pallas as pl
from jax.experimental.pallas import tpu as pltpu

N_TOK = 16384
D = 4096
E = 64
K = 8
M_BLK = 1024


def _router_block(x_ref, wt_ref, b_ref, rw_ref, gates_ref):
    acc = jnp.dot(x_ref[...], wt_ref[...], preferred_element_type=jnp.float32)
    rw = acc + b_ref[...]
    rw_ref[...] = rw

    cur = rw
    t = jnp.max(cur, axis=1, keepdims=True)
    m0 = t
    s = jnp.ones_like(t)
    for _ in range(K - 1):
        cur = jnp.where(cur == t, -jnp.inf, cur)
        t = jnp.max(cur, axis=1, keepdims=True)
        s = s + jnp.exp(t - m0)
    rinv = 1.0 / s
    gates_ref[...] = jnp.where(rw >= t, jnp.exp(rw - m0) * rinv, 0.0)


@jax.jit
def kernel(x, W, b):
    wt = W.T
    b2 = b.reshape(1, E)
    grid = (N_TOK // M_BLK,)
    rw, gates = pl.pallas_call(
        _router_block,
        grid=grid,
        in_specs=[
            pl.BlockSpec((M_BLK, D), lambda i: (i, 0)),
            pl.BlockSpec((D, E), lambda i: (0, 0)),
            pl.BlockSpec((1, E), lambda i: (0, 0)),
        ],
        out_specs=[
            pl.BlockSpec((M_BLK, E), lambda i: (i, 0)),
            pl.BlockSpec((M_BLK, E), lambda i: (i, 0)),
        ],
        out_shape=[
            jax.ShapeDtypeStruct((N_TOK, E), jnp.float32),
            jax.ShapeDtypeStruct((N_TOK, E), jnp.float32),
        ],
        compiler_params=pltpu.CompilerParams(
            dimension_semantics=("arbitrary",),
        ),
    )(x, wt, b2)
    return (gates, rw)
